# Initial kernel scaffold; baseline (speedup 1.0000x reference)
import functools

import jax
import jax.numpy as jnp
from jax import lax
from jax.experimental import pallas as pl
from jax.experimental.pallas import tpu as pltpu
from jax.experimental.pallas import tpu_sc as plsc

N = 10000
E = 320000
D = 128

NC = 2
NS = 16
NW = NC * NS

CHUNK = 128
NCHUNKS = E // CHUNK
CHUNKS_PER_W = NCHUNKS // NW
REM = NCHUNKS - CHUNKS_PER_W * NW
ROWS_PER_SUB = N // NS


def _sc_segment_sum(feat, src, dst):
    mesh = plsc.VectorSubcoreMesh(core_axis_name="c", subcore_axis_name="s")

    @functools.partial(
        pl.kernel,
        out_type=jax.ShapeDtypeStruct((NC * N, D), jnp.float32),
        mesh=mesh,
        scratch_types=[
            pltpu.VMEM((CHUNK,), jnp.int32),
            pltpu.VMEM((CHUNK,), jnp.int32),
            pltpu.VMEM((CHUNK, D), jnp.float32),
            pltpu.VMEM_SHARED((N, D), jnp.float32),
            pltpu.SemaphoreType.DMA,
        ],
    )
    def k(feat_hbm, src_hbm, dst_hbm, out_hbm, sidx, didx, rows, acc, sem):
        c = lax.axis_index("c")
        s = lax.axis_index("s")
        wid = c * NS + s

        @pl.loop(0, CHUNK)
        def _(i):
            @pl.loop(0, D, step=16)
            def _(j):
                rows.at[i, pl.ds(j, 16)][...] = jnp.zeros((16,), jnp.float32)

        base_r = s * ROWS_PER_SUB
        @pl.loop(0, 4)
        def _(r):
            pltpu.sync_copy(rows, acc.at[pl.ds(base_r + r * CHUNK, CHUNK)])
        pltpu.sync_copy(rows.at[pl.ds(0, 113)],
                        acc.at[pl.ds(base_r + 4 * CHUNK, 113)])
        plsc.subcore_barrier()

        def do_chunk(cid):
            e0 = cid * CHUNK
            pltpu.sync_copy(src_hbm.at[pl.ds(e0, CHUNK)], sidx)
            pltpu.sync_copy(dst_hbm.at[pl.ds(e0, CHUNK)], didx)
            pltpu.async_copy(feat_hbm.at[sidx], rows, sem).wait()
            pltpu.sync_copy(rows, acc.at[didx], add=True)

        base_c = wid * CHUNKS_PER_W

        @pl.loop(0, CHUNKS_PER_W)
        def _(t):
            do_chunk(base_c + t)

        @pl.when(wid < REM)
        def _():
            do_chunk(NW * CHUNKS_PER_W + wid)

        plsc.subcore_barrier()
        pltpu.sync_copy(acc.at[pl.ds(base_r, ROWS_PER_SUB)],
                        out_hbm.at[pl.ds(c * N + base_r, ROWS_PER_SUB)])

    return k(feat, src, dst)


def _tc_layer(x, p0, p1, W, b2d, final):
    BR = 1000

    def body(x_ref, p0_ref, p1_ref, w_ref, b_ref, o_ref):
        t = x_ref[...] + p0_ref[...] + p1_ref[...]
        acc = jnp.dot(t, w_ref[...], preferred_element_type=jnp.float32,
                      precision=lax.Precision.HIGHEST) + b_ref[...]
        if final:
            m = jnp.max(acc, axis=1, keepdims=True)
            e = acc - m
            lse = jnp.log(jnp.sum(jnp.exp(e), axis=1, keepdims=True))
            o_ref[...] = e - lse
        else:
            o_ref[...] = jnp.maximum(acc, 0.0)

    return pl.pallas_call(
        body,
        grid=(N // BR,),
        in_specs=[
            pl.BlockSpec((BR, D), lambda i: (i, 0)),
            pl.BlockSpec((BR, D), lambda i: (i, 0)),
            pl.BlockSpec((BR, D), lambda i: (i, 0)),
            pl.BlockSpec((D, D), lambda i: (0, 0)),
            pl.BlockSpec((1, D), lambda i: (0, 0)),
        ],
        out_specs=pl.BlockSpec((BR, D), lambda i: (i, 0)),
        out_shape=jax.ShapeDtypeStruct((N, D), jnp.float32),
    )(x, p0, p1, W, b2d)


def kernel(input_feature, edge_index, W1, b1, W2, b2):
    src = edge_index[0]
    dst = edge_index[1]
    b1_2d = b1.reshape(1, D)
    b2_2d = b2.reshape(1, D)

    p = _sc_segment_sum(input_feature, src, dst)
    h = _tc_layer(input_feature, p[:N], p[N:], W1, b1_2d, final=False)
    q = _sc_segment_sum(h, src, dst)
    return _tc_layer(h, q[:N], q[N:], W2, b2_2d, final=True)


# speedup vs baseline: 5.6361x; 5.6361x over previous
import functools

import jax
import jax.numpy as jnp
from jax import lax
from jax.experimental import pallas as pl
from jax.experimental.pallas import tpu as pltpu
from jax.experimental.pallas import tpu_sc as plsc

N = 10000
E = 320000
D = 128

NC = 2
NS = 16
NW = NC * NS

CHUNK = 128
NCHUNKS = E // CHUNK
CHUNKS_PER_W = NCHUNKS // NW
REM = NCHUNKS - CHUNKS_PER_W * NW
RPS = 632
RPS_LAST = N - RPS * (NS - 1)


def _sc_segment_sum(feat, src, dst):
    mesh = plsc.VectorSubcoreMesh(core_axis_name="c", subcore_axis_name="s")

    @functools.partial(
        pl.kernel,
        out_type=jax.ShapeDtypeStruct((NC * N, D), jnp.float32),
        mesh=mesh,
        scratch_types=[
            pltpu.VMEM((CHUNK,), jnp.int32),
            pltpu.VMEM((CHUNK,), jnp.int32),
            pltpu.VMEM((CHUNK, D), jnp.float32),
            pltpu.VMEM_SHARED((N, D), jnp.float32),
            pltpu.SemaphoreType.DMA,
        ],
    )
    def k(feat_hbm, src_hbm, dst_hbm, out_hbm, sidx, didx, rows, acc, sem):
        c = lax.axis_index("c")
        s = lax.axis_index("s")
        wid = c * NS + s

        @pl.loop(0, CHUNK)
        def _(i):
            @pl.loop(0, D, step=16)
            def _(j):
                rows.at[i, pl.ds(j, 16)][...] = jnp.zeros((16,), jnp.float32)

        base_r = s * RPS

        def zero_rows(tail):
            @pl.loop(0, 4)
            def _(r):
                pltpu.sync_copy(rows, acc.at[pl.ds(base_r + r * CHUNK, CHUNK)])
            pltpu.sync_copy(rows.at[pl.ds(0, tail)],
                            acc.at[pl.ds(base_r + 4 * CHUNK, tail)])

        @pl.when(s < NS - 1)
        def _():
            zero_rows(RPS - 4 * CHUNK)

        @pl.when(s == NS - 1)
        def _():
            zero_rows(RPS_LAST - 4 * CHUNK)

        plsc.subcore_barrier()

        def do_chunk(cid):
            e0 = cid * CHUNK
            pltpu.sync_copy(src_hbm.at[pl.ds(e0, CHUNK)], sidx)
            pltpu.sync_copy(dst_hbm.at[pl.ds(e0, CHUNK)], didx)
            pltpu.async_copy(feat_hbm.at[sidx], rows, sem).wait()
            pltpu.sync_copy(rows, acc.at[didx], add=True)

        base_c = wid * CHUNKS_PER_W

        @pl.loop(0, CHUNKS_PER_W)
        def _(t):
            do_chunk(base_c + t)

        @pl.when(wid < REM)
        def _():
            do_chunk(NW * CHUNKS_PER_W + wid)

        plsc.subcore_barrier()

        @pl.when(s < NS - 1)
        def _():
            pltpu.sync_copy(acc.at[pl.ds(base_r, RPS)],
                            out_hbm.at[pl.ds(c * N + base_r, RPS)])

        @pl.when(s == NS - 1)
        def _():
            pltpu.sync_copy(acc.at[pl.ds(base_r, RPS_LAST)],
                            out_hbm.at[pl.ds(c * N + base_r, RPS_LAST)])

    return k(feat, src, dst)


def _tc_layer(x, p0, p1, W, b2d, final):
    BR = 1000

    def body(x_ref, p0_ref, p1_ref, w_ref, b_ref, o_ref):
        t = x_ref[...] + p0_ref[...] + p1_ref[...]
        acc = jnp.dot(t, w_ref[...], preferred_element_type=jnp.float32,
                      precision=lax.Precision.HIGHEST) + b_ref[...]
        if final:
            m = jnp.max(acc, axis=1, keepdims=True)
            e = acc - m
            lse = jnp.log(jnp.sum(jnp.exp(e), axis=1, keepdims=True))
            o_ref[...] = e - lse
        else:
            o_ref[...] = jnp.maximum(acc, 0.0)

    return pl.pallas_call(
        body,
        grid=(N // BR,),
        in_specs=[
            pl.BlockSpec((BR, D), lambda i: (i, 0)),
            pl.BlockSpec((BR, D), lambda i: (i, 0)),
            pl.BlockSpec((BR, D), lambda i: (i, 0)),
            pl.BlockSpec((D, D), lambda i: (0, 0)),
            pl.BlockSpec((1, D), lambda i: (0, 0)),
        ],
        out_specs=pl.BlockSpec((BR, D), lambda i: (i, 0)),
        out_shape=jax.ShapeDtypeStruct((N, D), jnp.float32),
    )(x, p0, p1, W, b2d)


def kernel(input_feature, edge_index, W1, b1, W2, b2):
    src = edge_index[0]
    dst = edge_index[1]
    b1_2d = b1.reshape(1, D)
    b2_2d = b2.reshape(1, D)

    p = _sc_segment_sum(input_feature, src, dst)
    h = _tc_layer(input_feature, p[:N], p[N:], W1, b1_2d, final=False)
    q = _sc_segment_sum(h, src, dst)
    return _tc_layer(h, q[:N], q[N:], W2, b2_2d, final=True)
